# baseline (speedup 1.0000x reference)
import jax
import jax.numpy as jnp
from jax import lax
from jax.experimental import pallas as pl
from jax.experimental.pallas import tpu as pltpu
from jax.experimental.pallas import tpu_sc as plsc

_B, _L, _D = 1024, 200, 64
_NSEG = 3
_CH = 128
_NBUF = 5
_info = plsc.get_sparse_core_info()
_NC = _info.num_cores
_NW = _info.num_cores * _info.num_subcores
_ROWS_W = _B * _L // _NW
_NCH = _ROWS_W // _CH
_NSUP = _NCH // _NBUF
_LP = _L // _NBUF
_V = 100000
_VROWS_W = _V // _NW
_VCH = 125
_VNCH = _VROWS_W // _VCH


def _cast_body(tok_hbm, out_hbm, fbuf, bbuf):
    c = lax.axis_index("c")
    s = lax.axis_index("s")
    wid = s * _NC + c
    r0 = wid * _VROWS_W

    def chunk(ci, carry):
        base = r0 + ci * _VCH
        pltpu.sync_copy(tok_hbm.at[pl.ds(base, _VCH)], fbuf)

        def row(l, cr):
            for blk in range(_D // 32):
                a = fbuf[l, pl.ds(blk * 32, 16)]
                b2 = fbuf[l, pl.ds(blk * 32 + 16, 16)]
                bbuf[l, pl.ds(blk * 32, 32)] = plsc.pack(
                    a, b2, format=plsc.PackFormat.INTERLEAVED)
            return cr

        lax.fori_loop(0, _VCH, row, 0)
        pltpu.sync_copy(bbuf, out_hbm.at[pl.ds(base, _VCH)])
        return carry

    lax.fori_loop(0, _VNCH, chunk, 0)


def _body(seq_hbm, seg_hbm, tok_hbm, segtab_hbm, pe_hbm, out_hbm,
          idx_v, sidx_v, pe_v, segtab_v, peseg_sh, tok, ps, st, ss, so):
    c = lax.axis_index("c")
    s = lax.axis_index("s")
    wid = s * _NC + c
    r0 = wid * _NCH

    pltpu.sync_copy(seq_hbm.at[wid], idx_v)
    pltpu.sync_copy(seg_hbm.at[wid], sidx_v)

    iota = lax.iota(jnp.int32, 16)

    def comb_chunk(ci, cr):
        base = ci * _CH
        for k in range(_CH // 16):
            sl = pl.ds(base + k * 16, 16)
            lrow = lax.rem(base + k * 16 + iota, _L)
            sidx_v[sl] = lrow * _NSEG + sidx_v[sl]
        return cr

    lax.fori_loop(0, _NCH, comb_chunk, 0)

    @pl.when(s == 0)
    def _():
        pltpu.sync_copy(pe_hbm, pe_v)
        pltpu.sync_copy(segtab_hbm, segtab_v)
        for p in range(_NBUF):
            def build_row(l2, cr, p=p):
                l = p * _LP + l2
                for sg in range(_NSEG):
                    for q in range(_D // 16):
                        sl = pl.ds(q * 16, 16)
                        ps[0][l2 * _NSEG + sg, sl] = (pe_v[l, sl]
                                                      + segtab_v[sg, sl])
                return cr

            lax.fori_loop(0, _LP, build_row, 0)
            pltpu.sync_copy(ps[0].at[pl.ds(0, _LP * _NSEG)],
                            peseg_sh.at[pl.ds(p * _LP * _NSEG, _LP * _NSEG)])

    plsc.subcore_barrier()

    def issue(ci, k):
        isl = pl.ds(ci * _CH, _CH)
        pltpu.async_copy(tok_hbm.at[idx_v.at[isl]], tok[k], st[k])
        pltpu.async_copy(peseg_sh.at[sidx_v.at[isl]], ps[k], ss[k])

    def wait_g(k):
        isl = pl.ds(0, _CH)
        pltpu.make_async_copy(tok_hbm.at[idx_v.at[isl]], tok[k], st[k]).wait()
        pltpu.make_async_copy(peseg_sh.at[sidx_v.at[isl]], ps[k], ss[k]).wait()

    def wait_st(k):
        pltpu.make_async_copy(ps[k], out_hbm.at[pl.ds(0, _CH)],
                              so[k]).wait()

    def add_and_store(ci, k):
        def row(l, cr):
            for blk in range(_D // 32):
                x = tok[k][l, pl.ds(blk * 32, 32)]
                a, b2 = plsc.unpack(x, format=plsc.PackFormat.INTERLEAVED)
                sla = pl.ds(blk * 32, 16)
                slb = pl.ds(blk * 32 + 16, 16)
                ps[k][l, sla] = ps[k][l, sla] + a
                ps[k][l, slb] = ps[k][l, slb] + b2
            return cr

        lax.fori_loop(0, _CH, row, 0)
        pltpu.async_copy(ps[k], out_hbm.at[pl.ds((r0 + ci) * _CH, _CH)],
                         so[k])

    for k in range(_NBUF - 1):
        issue(k, k)

    def super_iter(su, carry):
        for k in range(_NBUF):
            ci = su * _NBUF + k
            prev = (k + _NBUF - 1) % _NBUF

            @pl.when(ci >= 1)
            def _(prev=prev):
                wait_st(prev)

            @pl.when(ci + _NBUF - 1 < _NCH)
            def _(ci=ci, prev=prev):
                issue(ci + _NBUF - 1, prev)

            wait_g(k)
            add_and_store(ci, k)
        return carry

    lax.fori_loop(0, _NSUP, super_iter, 0)
    wait_st((_NCH - 1) % _NBUF)


def kernel(sequence, segment_label, token_table, seg_table, pe):
    b, l = sequence.shape
    v, d = token_table.shape
    seqf = sequence.astype(jnp.int32).reshape(_NW, _ROWS_W)
    segf = segment_label.astype(jnp.int32).reshape(_NW, _ROWS_W)
    pe2 = pe[0, :l, :]
    cast = pl.kernel(
        _cast_body,
        out_type=jax.ShapeDtypeStruct((v, d), jnp.bfloat16),
        mesh=plsc.VectorSubcoreMesh(core_axis_name="c", subcore_axis_name="s"),
        compiler_params=pltpu.CompilerParams(use_tc_tiling_on_sc=False,
                                             needs_layout_passes=False),
        scratch_types=[
            pltpu.VMEM((_VCH, _D), jnp.float32),
            pltpu.VMEM((_VCH, _D), jnp.bfloat16),
        ],
    )
    tokb = cast(token_table)

    row_buf = pltpu.VMEM((_CH, _D), jnp.float32)
    row_buf16 = pltpu.VMEM((_CH, _D), jnp.bfloat16)
    k = pl.kernel(
        _body,
        out_type=jax.ShapeDtypeStruct((b * l, d), jnp.float32),
        mesh=plsc.VectorSubcoreMesh(core_axis_name="c", subcore_axis_name="s"),
        compiler_params=pltpu.CompilerParams(use_tc_tiling_on_sc=False,
                                             needs_layout_passes=False),
        scratch_types=[
            pltpu.VMEM((_ROWS_W,), jnp.int32),
            pltpu.VMEM((_ROWS_W,), jnp.int32),
            pltpu.VMEM((_L, _D), jnp.float32),
            pltpu.VMEM((_NSEG, _D), jnp.float32),
            pltpu.VMEM_SHARED((_L * _NSEG, _D), jnp.float32),
            [row_buf16] * _NBUF,
            [row_buf] * _NBUF,
            [pltpu.SemaphoreType.DMA] * _NBUF,
            [pltpu.SemaphoreType.DMA] * _NBUF,
            [pltpu.SemaphoreType.DMA] * _NBUF,
        ],
    )
    out = k(seqf, segf, tokb, seg_table, pe2)
    return out.reshape(b, l, d)


# speedup vs baseline: 1.4177x; 1.4177x over previous
import jax
import jax.numpy as jnp
from jax import lax
from jax.experimental import pallas as pl
from jax.experimental.pallas import tpu as pltpu
from jax.experimental.pallas import tpu_sc as plsc

_B, _L, _D = 1024, 200, 64
_NSEG = 3
_CH = 128
_NBUF = 5
_info = plsc.get_sparse_core_info()
_NC = _info.num_cores
_NW = _info.num_cores * _info.num_subcores
_ROWS_W = _B * _L // _NW
_NCH = _ROWS_W // _CH
_NSUP = _NCH // _NBUF
_LP = _L // _NBUF


def _body(seq_hbm, seg_hbm, tok_hbm, segtab_hbm, pe_hbm, out_hbm,
          idx_v, sidx_v, pe_v, segtab_v, peseg_sh, tok, ps, st, ss, so):
    c = lax.axis_index("c")
    s = lax.axis_index("s")
    wid = s * _NC + c
    r0 = wid * _NCH

    pltpu.sync_copy(seq_hbm.at[wid], idx_v)
    pltpu.sync_copy(seg_hbm.at[wid], sidx_v)

    iota = lax.iota(jnp.int32, 16)

    def comb_chunk(ci, cr):
        base = ci * _CH
        for k in range(_CH // 16):
            sl = pl.ds(base + k * 16, 16)
            lrow = lax.rem(base + k * 16 + iota, _L)
            sidx_v[sl] = lrow * _NSEG + sidx_v[sl]
        return cr

    lax.fori_loop(0, _NCH, comb_chunk, 0)

    @pl.when(s == 0)
    def _():
        pltpu.sync_copy(pe_hbm, pe_v)
        pltpu.sync_copy(segtab_hbm, segtab_v)
        for p in range(_NBUF):
            def build_row(l2, cr, p=p):
                l = p * _LP + l2
                for sg in range(_NSEG):
                    for q in range(_D // 16):
                        sl = pl.ds(q * 16, 16)
                        ps[0][l2 * _NSEG + sg, sl] = (pe_v[l, sl]
                                                      + segtab_v[sg, sl])
                return cr

            lax.fori_loop(0, _LP, build_row, 0)
            pltpu.sync_copy(ps[0].at[pl.ds(0, _LP * _NSEG)],
                            peseg_sh.at[pl.ds(p * _LP * _NSEG, _LP * _NSEG)])

    plsc.subcore_barrier()

    def issue(ci, k):
        isl = pl.ds(ci * _CH, _CH)
        pltpu.async_copy(tok_hbm.at[idx_v.at[isl]], tok[k], st[k])
        pltpu.async_copy(peseg_sh.at[sidx_v.at[isl]], ps[k], ss[k])

    def wait_g(k):
        isl = pl.ds(0, _CH)
        pltpu.make_async_copy(tok_hbm.at[idx_v.at[isl]], tok[k], st[k]).wait()
        pltpu.make_async_copy(peseg_sh.at[sidx_v.at[isl]], ps[k], ss[k]).wait()

    def wait_st(k):
        pltpu.make_async_copy(ps[k], out_hbm.at[pl.ds(0, _CH)],
                              so[k]).wait()

    def add_and_store(ci, k):
        def row(l, cr):
            for q in range(_D // 16):
                sl = pl.ds(q * 16, 16)
                ps[k][l, sl] = ps[k][l, sl] + tok[k][l, sl]
            return cr

        lax.fori_loop(0, _CH, row, 0)
        pltpu.async_copy(ps[k], out_hbm.at[pl.ds((r0 + ci) * _CH, _CH)],
                         so[k])

    for k in range(_NBUF - 1):
        issue(k, k)

    def super_iter(su, carry):
        for k in range(_NBUF):
            ci = su * _NBUF + k
            prev = (k + _NBUF - 1) % _NBUF

            @pl.when(ci >= 1)
            def _(prev=prev):
                wait_st(prev)

            @pl.when(ci + _NBUF - 1 < _NCH)
            def _(ci=ci, prev=prev):
                issue(ci + _NBUF - 1, prev)

            wait_g(k)
            add_and_store(ci, k)
        return carry

    lax.fori_loop(0, _NSUP, super_iter, 0)
    wait_st((_NCH - 1) % _NBUF)


def kernel(sequence, segment_label, token_table, seg_table, pe):
    b, l = sequence.shape
    v, d = token_table.shape
    seqf = sequence.astype(jnp.int32).reshape(_NW, _ROWS_W)
    segf = segment_label.astype(jnp.int32).reshape(_NW, _ROWS_W)
    pe2 = pe[0, :l, :]
    row_buf = pltpu.VMEM((_CH, _D), jnp.float32)
    k = pl.kernel(
        _body,
        out_type=jax.ShapeDtypeStruct((b * l, d), jnp.float32),
        mesh=plsc.VectorSubcoreMesh(core_axis_name="c", subcore_axis_name="s"),
        compiler_params=pltpu.CompilerParams(use_tc_tiling_on_sc=False,
                                             needs_layout_passes=False),
        scratch_types=[
            pltpu.VMEM((_ROWS_W,), jnp.int32),
            pltpu.VMEM((_ROWS_W,), jnp.int32),
            pltpu.VMEM((_L, _D), jnp.float32),
            pltpu.VMEM((_NSEG, _D), jnp.float32),
            pltpu.VMEM_SHARED((_L * _NSEG, _D), jnp.float32),
            [row_buf] * _NBUF,
            [row_buf] * _NBUF,
            [pltpu.SemaphoreType.DMA] * _NBUF,
            [pltpu.SemaphoreType.DMA] * _NBUF,
            [pltpu.SemaphoreType.DMA] * _NBUF,
        ],
    )
    out = k(seqf, segf, token_table, seg_table, pe2)
    return out.reshape(b, l, d)
